# Initial kernel scaffold; baseline (speedup 1.0000x reference)
import jax
import jax.numpy as jnp
from jax.experimental import pallas as pl


def kernel(atom_feas, bond_feas, bond_weights, atom_graph, directed2undirected, W1c, b1c, W2c, b2c, W1g, b1g, W2g, b2g, Wout, bout):
    raise NotImplementedError("write your pallas kernel here")


# speedup vs baseline: 3.0550x; 3.0550x over previous
import functools

import jax
import jax.numpy as jnp
from jax import lax
from jax.experimental import pallas as pl
from jax.experimental.pallas import tpu as pltpu
from jax.experimental.pallas import tpu_sc as plsc

N_ATOMS = 10000
N_DIR = 320000
N_UND = 160000
ATOM_DIM = 128
HIDDEN = 64

NW = 32
C = 128
NCHUNK = 79
E_W = NCHUNK * C
N_PAD = NW * E_W
N_ACC = 10112
ROWS_PER_TILE = N_ACC // 16

_mesh = plsc.VectorSubcoreMesh(core_axis_name="c", subcore_axis_name="s")


def _ptables_body(af_ref, wctr_ref, wnbr_ref, pc_ref, pn_ref):
    af = af_ref[...]
    pc_ref[...] = jnp.dot(af, wctr_ref[...], preferred_element_type=jnp.float32)
    pn_ref[...] = jnp.dot(af, wnbr_ref[...], preferred_element_type=jnp.float32)


def _btable_body(bf_ref, wb_ref, bb_ref, out_ref):
    out_ref[...] = (
        jnp.dot(bf_ref[...], wb_ref[...], preferred_element_type=jnp.float32)
        + bb_ref[...]
    )


def _gather_h_body(pctr, pnbr, btab, cent, nbr, und, h_out,
                   cidx, nidx, uidx, bc, bn, bb, s1, s2, s3):
    wid = lax.axis_index("s") * 2 + lax.axis_index("c")
    base = wid * E_W

    def chunk(k, carry):
        off = base + k * C
        pltpu.sync_copy(cent.at[pl.ds(off, C)], cidx)
        pltpu.sync_copy(nbr.at[pl.ds(off, C)], nidx)
        pltpu.sync_copy(und.at[pl.ds(off, C)], uidx)
        d1 = pltpu.async_copy(pctr.at[cidx], bc, s1)
        d2 = pltpu.async_copy(pnbr.at[nidx], bn, s2)
        d3 = pltpu.async_copy(btab.at[uidx], bb, s3)
        d1.wait()
        d2.wait()
        d3.wait()

        def row(r, rc):
            for j in range(ATOM_DIM // 16):
                sl = (r, pl.ds(j * 16, 16))
                bb[sl] = bb[sl] + bc[sl] + bn[sl]
            return rc

        lax.fori_loop(0, C, row, 0)
        pltpu.sync_copy(bb, h_out.at[pl.ds(off, C)])
        return carry

    lax.fori_loop(0, NCHUNK, chunk, 0)


_gather_h = functools.partial(
    pl.kernel,
    out_type=jax.ShapeDtypeStruct((N_PAD, ATOM_DIM), jnp.float32),
    mesh=_mesh,
    scratch_types=[
        pltpu.VMEM((C,), jnp.int32),
        pltpu.VMEM((C,), jnp.int32),
        pltpu.VMEM((C,), jnp.int32),
        pltpu.VMEM((C, ATOM_DIM), jnp.float32),
        pltpu.VMEM((C, ATOM_DIM), jnp.float32),
        pltpu.VMEM((C, ATOM_DIM), jnp.float32),
        pltpu.SemaphoreType.DMA,
        pltpu.SemaphoreType.DMA,
        pltpu.SemaphoreType.DMA,
    ],
)(_gather_h_body)


def _mlp_body(h_ref, w2_ref, b2_ref, o_ref):
    h = h_ref[...]
    s = h * jax.nn.sigmoid(h)
    t = jnp.dot(s, w2_ref[...], preferred_element_type=jnp.float32) + b2_ref[...]
    core = t[:, :ATOM_DIM]
    gate = t[:, ATOM_DIM:]
    o_ref[...] = core * jax.nn.sigmoid(core) * jax.nn.sigmoid(gate)


def _scatter_body(msg, bwt, und, cent, zeros, out,
                  uidx, cidx, mbuf, wbuf, acc, s1, s2):
    cid = lax.axis_index("c")
    sid = lax.axis_index("s")
    wid = sid * 2 + cid
    r0 = sid * ROWS_PER_TILE
    pltpu.sync_copy(zeros.at[pl.ds(r0, ROWS_PER_TILE)],
                    acc.at[pl.ds(r0, ROWS_PER_TILE)])
    plsc.subcore_barrier()
    base = wid * E_W

    def chunk(k, carry):
        off = base + k * C
        pltpu.sync_copy(und.at[pl.ds(off, C)], uidx)
        pltpu.sync_copy(cent.at[pl.ds(off, C)], cidx)
        d1 = pltpu.async_copy(msg.at[pl.ds(off, C)], mbuf, s1)
        d2 = pltpu.async_copy(bwt.at[uidx], wbuf, s2)
        d1.wait()
        d2.wait()

        def row(r, rc):
            for j in range(ATOM_DIM // 16):
                sl = (r, pl.ds(j * 16, 16))
                mbuf[sl] = mbuf[sl] * wbuf[sl]
            return rc

        lax.fori_loop(0, C, row, 0)
        pltpu.sync_copy(mbuf, acc.at[cidx], add=True)
        return carry

    lax.fori_loop(0, NCHUNK, chunk, 0)
    plsc.subcore_barrier()
    pltpu.sync_copy(acc.at[pl.ds(r0, ROWS_PER_TILE)],
                    out.at[cid, pl.ds(r0, ROWS_PER_TILE)])


_scatter = functools.partial(
    pl.kernel,
    out_type=jax.ShapeDtypeStruct((2, N_ACC, ATOM_DIM), jnp.float32),
    mesh=_mesh,
    scratch_types=[
        pltpu.VMEM((C,), jnp.int32),
        pltpu.VMEM((C,), jnp.int32),
        pltpu.VMEM((C, ATOM_DIM), jnp.float32),
        pltpu.VMEM((C, ATOM_DIM), jnp.float32),
        pltpu.VMEM_SHARED((N_ACC, ATOM_DIM), jnp.float32),
        pltpu.SemaphoreType.DMA,
        pltpu.SemaphoreType.DMA,
    ],
)(_scatter_body)


def _final_body(p0_ref, p1_ref, wout_ref, bout_ref, af_ref, o_ref):
    a = p0_ref[...] + p1_ref[...]
    o_ref[...] = (
        jnp.dot(a, wout_ref[...], preferred_element_type=jnp.float32)
        + bout_ref[...]
        + af_ref[...]
    )


def kernel(atom_feas, bond_feas, bond_weights, atom_graph, directed2undirected,
           W1c, b1c, W2c, b2c, W1g, b1g, W2g, b2g, Wout, bout):
    f32 = jnp.float32
    Wctr = jnp.concatenate([W1c[:ATOM_DIM], W1g[:ATOM_DIM]], axis=1)
    Wnbr = jnp.concatenate([W1c[ATOM_DIM + 16:], W1g[ATOM_DIM + 16:]], axis=1)
    Wbnd = jnp.concatenate([W1c[ATOM_DIM:ATOM_DIM + 16],
                            W1g[ATOM_DIM:ATOM_DIM + 16]], axis=1)
    bcat1 = jnp.concatenate([b1c, b1g])[None, :]
    W2blk = jnp.zeros((ATOM_DIM, 2 * ATOM_DIM), f32)
    W2blk = W2blk.at[:HIDDEN, :ATOM_DIM].set(W2c)
    W2blk = W2blk.at[HIDDEN:, ATOM_DIM:].set(W2g)
    bcat2 = jnp.concatenate([b2c, b2g])[None, :]

    pad = N_PAD - N_DIR
    cent = jnp.concatenate(
        [atom_graph[:, 0], jnp.full((pad,), N_ATOMS, jnp.int32)])
    nbrs = jnp.concatenate([atom_graph[:, 1], jnp.zeros((pad,), jnp.int32)])
    und = jnp.concatenate([directed2undirected, jnp.zeros((pad,), jnp.int32)])
    af_pad = jnp.concatenate(
        [atom_feas, jnp.zeros((N_ACC - N_ATOMS, ATOM_DIM), f32)])

    pctr, pnbr = pl.pallas_call(
        _ptables_body,
        out_shape=[
            jax.ShapeDtypeStruct((N_ACC, ATOM_DIM), f32),
            jax.ShapeDtypeStruct((N_ACC, ATOM_DIM), f32),
        ],
    )(af_pad, Wctr, Wnbr)

    btab = pl.pallas_call(
        _btable_body,
        grid=(20,),
        in_specs=[
            pl.BlockSpec((N_UND // 20, 16), lambda i: (i, 0)),
            pl.BlockSpec((16, ATOM_DIM), lambda i: (0, 0)),
            pl.BlockSpec((1, ATOM_DIM), lambda i: (0, 0)),
        ],
        out_specs=pl.BlockSpec((N_UND // 20, ATOM_DIM), lambda i: (i, 0)),
        out_shape=jax.ShapeDtypeStruct((N_UND, ATOM_DIM), f32),
    )(bond_feas, Wbnd, bcat1)

    h = _gather_h(pctr, pnbr, btab, cent, nbrs, und)

    BLK = 4096
    msg = pl.pallas_call(
        _mlp_body,
        grid=(N_PAD // BLK,),
        in_specs=[
            pl.BlockSpec((BLK, ATOM_DIM), lambda i: (i, 0)),
            pl.BlockSpec((ATOM_DIM, 2 * ATOM_DIM), lambda i: (0, 0)),
            pl.BlockSpec((1, 2 * ATOM_DIM), lambda i: (0, 0)),
        ],
        out_specs=pl.BlockSpec((BLK, ATOM_DIM), lambda i: (i, 0)),
        out_shape=jax.ShapeDtypeStruct((N_PAD, ATOM_DIM), f32),
    )(h, W2blk, bcat2)

    zeros = jnp.zeros((N_ACC, ATOM_DIM), f32)
    partials = _scatter(msg, bond_weights, und, cent, zeros)

    out = pl.pallas_call(
        _final_body,
        out_shape=jax.ShapeDtypeStruct((N_ATOMS, ATOM_DIM), f32),
    )(partials[0, :N_ATOMS], partials[1, :N_ATOMS], Wout, bout[None, :],
      atom_feas)
    return out


# speedup vs baseline: 4.2868x; 1.4032x over previous
import functools

import jax
import jax.numpy as jnp
from jax import lax
from jax.experimental import pallas as pl
from jax.experimental.pallas import tpu as pltpu
from jax.experimental.pallas import tpu_sc as plsc

N_ATOMS = 10000
N_DIR = 320000
N_UND = 160000
ATOM_DIM = 128
HIDDEN = 64

NW = 32
C = 64
NCHUNK = 158
E_W = NCHUNK * C
N_PAD = NW * E_W
N_ACC = 10112
ROWS_PER_TILE = N_ACC // 16

_mesh = plsc.VectorSubcoreMesh(core_axis_name="c", subcore_axis_name="s")


def _ptables_body(af_ref, wctr_ref, wnbr_ref, pc_ref, pn_ref):
    af = af_ref[...]
    pc_ref[...] = jnp.dot(af, wctr_ref[...], preferred_element_type=jnp.float32)
    pn_ref[...] = jnp.dot(af, wnbr_ref[...], preferred_element_type=jnp.float32)


def _btable_body(bf_ref, wb_ref, bb_ref, out_ref):
    out_ref[...] = (
        jnp.dot(bf_ref[...], wb_ref[...], preferred_element_type=jnp.float32)
        + bb_ref[...]
    )


def _gather_h_body(pctr, pnbr, btab, cent2, nbr2, und2, h_out,
                   cia, nia, uia,
                   bc0, bn0, bb0, o0, bc1, bn1, bb1, o1,
                   g0, g1, s0, s1):
    wid = lax.axis_index("s") * 2 + lax.axis_index("c")
    base = wid * E_W
    pltpu.sync_copy(cent2.at[wid], cia)
    pltpu.sync_copy(nbr2.at[wid], nia)
    pltpu.sync_copy(und2.at[wid], uia)
    sets = ((bc0, bn0, bb0, o0, g0, s0), (bc1, bn1, bb1, o1, g1, s1))

    def fire(q, st):
        bc, bn, bb, _, g, _ = st
        pltpu.async_copy(pctr.at[cia.at[pl.ds(q * C, C)]], bc, g)
        pltpu.async_copy(pnbr.at[nia.at[pl.ds(q * C, C)]], bn, g)
        pltpu.async_copy(btab.at[uia.at[pl.ds(q * C, C)]], bb, g)

    def wait_gathers(q, st):
        bc, bn, bb, _, g, _ = st
        pltpu.make_async_copy(pctr.at[cia.at[pl.ds(q * C, C)]], bc, g).wait()
        pltpu.make_async_copy(pnbr.at[nia.at[pl.ds(q * C, C)]], bn, g).wait()
        pltpu.make_async_copy(btab.at[uia.at[pl.ds(q * C, C)]], bb, g).wait()

    fire(0, sets[0])

    def body(k, carry):
        for b in (0, 1):
            q = 2 * k + b
            st = sets[b]
            bc, bn, bb, o, g, s = st

            @pl.when(q + 1 < NCHUNK)
            def _():
                fire(q + 1, sets[1 - b])

            wait_gathers(q, st)

            @pl.when(q >= 2)
            def _():
                pltpu.make_async_copy(
                    o, h_out.at[pl.ds(base + (q - 2) * C, C)], s).wait()

            def row(r, rc):
                for j in range(ATOM_DIM // 16):
                    sl = (r, pl.ds(j * 16, 16))
                    o[sl] = bc[sl] + bn[sl] + bb[sl]
                return rc

            lax.fori_loop(0, C, row, 0)
            pltpu.async_copy(o, h_out.at[pl.ds(base + q * C, C)], s)
        return carry

    lax.fori_loop(0, NCHUNK // 2, body, 0)
    pltpu.make_async_copy(
        o0, h_out.at[pl.ds(base + (NCHUNK - 2) * C, C)], s0).wait()
    pltpu.make_async_copy(
        o1, h_out.at[pl.ds(base + (NCHUNK - 1) * C, C)], s1).wait()


_gather_h = functools.partial(
    pl.kernel,
    out_type=jax.ShapeDtypeStruct((N_PAD, ATOM_DIM), jnp.float32),
    mesh=_mesh,
    scratch_types=[
        pltpu.VMEM((E_W,), jnp.int32),
        pltpu.VMEM((E_W,), jnp.int32),
        pltpu.VMEM((E_W,), jnp.int32),
        pltpu.VMEM((C, ATOM_DIM), jnp.float32),
        pltpu.VMEM((C, ATOM_DIM), jnp.float32),
        pltpu.VMEM((C, ATOM_DIM), jnp.float32),
        pltpu.VMEM((C, ATOM_DIM), jnp.float32),
        pltpu.VMEM((C, ATOM_DIM), jnp.float32),
        pltpu.VMEM((C, ATOM_DIM), jnp.float32),
        pltpu.VMEM((C, ATOM_DIM), jnp.float32),
        pltpu.VMEM((C, ATOM_DIM), jnp.float32),
        pltpu.SemaphoreType.DMA,
        pltpu.SemaphoreType.DMA,
        pltpu.SemaphoreType.DMA,
        pltpu.SemaphoreType.DMA,
    ],
)(_gather_h_body)


def _mlp_body(h_ref, w2_ref, b2_ref, o_ref):
    h = h_ref[...]
    s = h * jax.nn.sigmoid(h)
    t = jnp.dot(s, w2_ref[...], preferred_element_type=jnp.float32) + b2_ref[...]
    core = t[:, :ATOM_DIM]
    gate = t[:, ATOM_DIM:]
    o_ref[...] = core * jax.nn.sigmoid(core) * jax.nn.sigmoid(gate)


def _scatter_body(msg, bwt, und2, cent2, zeros, out,
                  uia, cia, m0, w0, m1, w1, acc, g0, g1, sc0, sc1, ci0, ci1):
    cid = lax.axis_index("c")
    sid = lax.axis_index("s")
    wid = sid * 2 + cid
    r0 = sid * ROWS_PER_TILE
    pltpu.sync_copy(zeros.at[pl.ds(r0, ROWS_PER_TILE)],
                    acc.at[pl.ds(r0, ROWS_PER_TILE)])
    plsc.subcore_barrier()
    base = wid * E_W
    pltpu.sync_copy(und2.at[wid], uia)
    pltpu.sync_copy(cent2.at[wid, pl.ds(0, C)], cia.at[0])
    sets = ((m0, w0, g0, sc0, ci0), (m1, w1, g1, sc1, ci1))

    def fire(q, st):
        m, w, g, _, _ = st
        pltpu.async_copy(msg.at[pl.ds(base + q * C, C)], m, g)
        pltpu.async_copy(bwt.at[uia.at[pl.ds(q * C, C)]], w, g)

    def wait_gathers(q, st):
        m, w, g, _, _ = st
        pltpu.make_async_copy(msg.at[pl.ds(base + q * C, C)], m, g).wait()
        pltpu.make_async_copy(bwt.at[uia.at[pl.ds(q * C, C)]], w, g).wait()

    fire(0, sets[0])

    def body(k, carry):
        for b in (0, 1):
            q = 2 * k + b
            m, w, g, sc, ci = sets[b]
            mo, wo, go, sco, cio = sets[1 - b]

            @pl.when(q >= 1)
            def _():
                pltpu.make_async_copy(mo, acc.at[cia.at[1 - b]], sco).wait()

            @pl.when(q + 1 < NCHUNK)
            def _():
                fire(q + 1, sets[1 - b])
                pltpu.async_copy(cent2.at[wid, pl.ds((q + 1) * C, C)],
                                 cia.at[1 - b], cio)

            wait_gathers(q, sets[b])

            def row(r, rc):
                for j in range(ATOM_DIM // 16):
                    sl = (r, pl.ds(j * 16, 16))
                    m[sl] = m[sl] * w[sl]
                return rc

            lax.fori_loop(0, C, row, 0)

            @pl.when(q >= 1)
            def _():
                pltpu.make_async_copy(cent2.at[wid, pl.ds(q * C, C)],
                                      cia.at[b], ci).wait()

            pltpu.async_copy(m, acc.at[cia.at[b]], sc, add=True)
        return carry

    lax.fori_loop(0, NCHUNK // 2, body, 0)
    pltpu.make_async_copy(m1, acc.at[cia.at[1]], sc1).wait()
    plsc.subcore_barrier()
    pltpu.sync_copy(acc.at[pl.ds(r0, ROWS_PER_TILE)],
                    out.at[cid, pl.ds(r0, ROWS_PER_TILE)])


_scatter = functools.partial(
    pl.kernel,
    out_type=jax.ShapeDtypeStruct((2, N_ACC, ATOM_DIM), jnp.float32),
    mesh=_mesh,
    scratch_types=[
        pltpu.VMEM((E_W,), jnp.int32),
        pltpu.VMEM((2, C), jnp.int32),
        pltpu.VMEM((C, ATOM_DIM), jnp.float32),
        pltpu.VMEM((C, ATOM_DIM), jnp.float32),
        pltpu.VMEM((C, ATOM_DIM), jnp.float32),
        pltpu.VMEM((C, ATOM_DIM), jnp.float32),
        pltpu.VMEM_SHARED((N_ACC, ATOM_DIM), jnp.float32),
        pltpu.SemaphoreType.DMA,
        pltpu.SemaphoreType.DMA,
        pltpu.SemaphoreType.DMA,
        pltpu.SemaphoreType.DMA,
        pltpu.SemaphoreType.DMA,
        pltpu.SemaphoreType.DMA,
    ],
)(_scatter_body)


def _final_body(p0_ref, p1_ref, wout_ref, bout_ref, af_ref, o_ref):
    a = p0_ref[...] + p1_ref[...]
    o_ref[...] = (
        jnp.dot(a, wout_ref[...], preferred_element_type=jnp.float32)
        + bout_ref[...]
        + af_ref[...]
    )


def kernel(atom_feas, bond_feas, bond_weights, atom_graph, directed2undirected,
           W1c, b1c, W2c, b2c, W1g, b1g, W2g, b2g, Wout, bout):
    f32 = jnp.float32
    Wctr = jnp.concatenate([W1c[:ATOM_DIM], W1g[:ATOM_DIM]], axis=1)
    Wnbr = jnp.concatenate([W1c[ATOM_DIM + 16:], W1g[ATOM_DIM + 16:]], axis=1)
    Wbnd = jnp.concatenate([W1c[ATOM_DIM:ATOM_DIM + 16],
                            W1g[ATOM_DIM:ATOM_DIM + 16]], axis=1)
    bcat1 = jnp.concatenate([b1c, b1g])[None, :]
    W2blk = jnp.zeros((ATOM_DIM, 2 * ATOM_DIM), f32)
    W2blk = W2blk.at[:HIDDEN, :ATOM_DIM].set(W2c)
    W2blk = W2blk.at[HIDDEN:, ATOM_DIM:].set(W2g)
    bcat2 = jnp.concatenate([b2c, b2g])[None, :]

    pad = N_PAD - N_DIR
    cent = jnp.concatenate(
        [atom_graph[:, 0], jnp.full((pad,), N_ATOMS, jnp.int32)])
    nbrs = jnp.concatenate([atom_graph[:, 1], jnp.zeros((pad,), jnp.int32)])
    und = jnp.concatenate([directed2undirected, jnp.zeros((pad,), jnp.int32)])
    cent2 = cent.reshape(NW, E_W)
    nbr2 = nbrs.reshape(NW, E_W)
    und2 = und.reshape(NW, E_W)
    af_pad = jnp.concatenate(
        [atom_feas, jnp.zeros((N_ACC - N_ATOMS, ATOM_DIM), f32)])

    pctr, pnbr = pl.pallas_call(
        _ptables_body,
        out_shape=[
            jax.ShapeDtypeStruct((N_ACC, ATOM_DIM), f32),
            jax.ShapeDtypeStruct((N_ACC, ATOM_DIM), f32),
        ],
    )(af_pad, Wctr, Wnbr)

    btab = pl.pallas_call(
        _btable_body,
        grid=(20,),
        in_specs=[
            pl.BlockSpec((N_UND // 20, 16), lambda i: (i, 0)),
            pl.BlockSpec((16, ATOM_DIM), lambda i: (0, 0)),
            pl.BlockSpec((1, ATOM_DIM), lambda i: (0, 0)),
        ],
        out_specs=pl.BlockSpec((N_UND // 20, ATOM_DIM), lambda i: (i, 0)),
        out_shape=jax.ShapeDtypeStruct((N_UND, ATOM_DIM), f32),
    )(bond_feas, Wbnd, bcat1)

    h = _gather_h(pctr, pnbr, btab, cent2, nbr2, und2)

    BLK = 4096
    msg = pl.pallas_call(
        _mlp_body,
        grid=(N_PAD // BLK,),
        in_specs=[
            pl.BlockSpec((BLK, ATOM_DIM), lambda i: (i, 0)),
            pl.BlockSpec((ATOM_DIM, 2 * ATOM_DIM), lambda i: (0, 0)),
            pl.BlockSpec((1, 2 * ATOM_DIM), lambda i: (0, 0)),
        ],
        out_specs=pl.BlockSpec((BLK, ATOM_DIM), lambda i: (i, 0)),
        out_shape=jax.ShapeDtypeStruct((N_PAD, ATOM_DIM), f32),
    )(h, W2blk, bcat2)

    zeros = jnp.zeros((N_ACC, ATOM_DIM), f32)
    partials = _scatter(msg, bond_weights, und2, cent2, zeros)

    out = pl.pallas_call(
        _final_body,
        out_shape=jax.ShapeDtypeStruct((N_ATOMS, ATOM_DIM), f32),
    )(partials[0, :N_ATOMS], partials[1, :N_ATOMS], Wout, bout[None, :],
      atom_feas)
    return out
